# Initial kernel scaffold; baseline (speedup 1.0000x reference)
import jax
import jax.numpy as jnp
from jax.experimental import pallas as pl


def kernel(x, table, W1, b1, W2, b2):
    raise NotImplementedError("write your pallas kernel here")


# speedup vs baseline: 3.1976x; 3.1976x over previous
import functools

import jax
import jax.numpy as jnp
from jax import lax
from jax.experimental import pallas as pl
from jax.experimental.pallas import tpu as pltpu
from jax.experimental.pallas import tpu_sc as plsc

EMBED_DIM = 64
HIDDEN_DIM = 128
OUT_DIM = 32
BATCH = 16384
HIST = 200

NUM_WORKERS = 32
E_PER_W = BATCH // NUM_WORKERS
CHUNK = 4
ROWS = CHUNK * HIST
NCHUNK = E_PER_W // CHUNK
INV_H = 1.0 / HIST

_mesh = plsc.VectorSubcoreMesh(core_axis_name="c", subcore_axis_name="s")


@functools.partial(
    pl.kernel,
    mesh=_mesh,
    out_type=jax.ShapeDtypeStruct((BATCH, EMBED_DIM), jnp.float32),
    scratch_types=[
        pltpu.VMEM((ROWS,), jnp.int32),
        pltpu.VMEM((ROWS,), jnp.int32),
        pltpu.VMEM((ROWS, EMBED_DIM), jnp.float32),
        pltpu.VMEM((ROWS, EMBED_DIM), jnp.float32),
        pltpu.VMEM((CHUNK, EMBED_DIM), jnp.float32),
        pltpu.SemaphoreType.DMA,
        pltpu.SemaphoreType.DMA,
    ],
    compiler_params=pltpu.CompilerParams(use_tc_tiling_on_sc=False),
)
def _pool(x_hbm, table_hbm, out_hbm, idx0, idx1, rb0, rb1, stage, sem0, sem1):
    wid = lax.axis_index("s") * 2 + lax.axis_index("c")
    flat_base = wid * (E_PER_W * HIST)
    out_base = wid * E_PER_W

    def start(c, ibuf, rbuf, sem):
        pltpu.sync_copy(x_hbm.at[pl.ds(flat_base + c * ROWS, ROWS)], ibuf)
        pltpu.make_async_copy(table_hbm.at[ibuf], rbuf, sem).start()

    def finish_and_accum(c, ibuf, rbuf, sem):
        pltpu.make_async_copy(table_hbm.at[ibuf], rbuf, sem).wait()
        for e in range(CHUNK):
            base_r = e * HIST

            def body(i, accs, base_r=base_r, rbuf=rbuf):
                a0, a1, a2, a3 = accs
                r = base_r + i
                a0 = a0 + rbuf[r, pl.ds(0, 16)]
                a1 = a1 + rbuf[r, pl.ds(16, 16)]
                a2 = a2 + rbuf[r, pl.ds(32, 16)]
                a3 = a3 + rbuf[r, pl.ds(48, 16)]
                return (a0, a1, a2, a3)

            z = jnp.zeros((16,), jnp.float32)
            a0, a1, a2, a3 = lax.fori_loop(0, HIST, body, (z, z, z, z),
                                           unroll=8)
            stage[e, pl.ds(0, 16)] = a0 * INV_H
            stage[e, pl.ds(16, 16)] = a1 * INV_H
            stage[e, pl.ds(32, 16)] = a2 * INV_H
            stage[e, pl.ds(48, 16)] = a3 * INV_H
        pltpu.sync_copy(stage, out_hbm.at[pl.ds(out_base + c * CHUNK, CHUNK)])

    start(0, idx0, rb0, sem0)

    def pair(p, carry):
        c0 = 2 * p
        start(c0 + 1, idx1, rb1, sem1)
        finish_and_accum(c0, idx0, rb0, sem0)

        @pl.when(p < NCHUNK // 2 - 1)
        def _():
            start(c0 + 2, idx0, rb0, sem0)

        finish_and_accum(c0 + 1, idx1, rb1, sem1)
        return carry

    lax.fori_loop(0, NCHUNK // 2, pair, 0)


def _mlp_body(m_ref, w1_ref, b1_ref, w2_ref, b2_ref, o_ref):
    m = m_ref[...]
    h = lax.dot_general(m, w1_ref[...], (((1,), (0,)), ((), ())),
                        preferred_element_type=jnp.float32)
    h = jnp.maximum(h + b1_ref[...], 0.0)
    o = lax.dot_general(h, w2_ref[...], (((1,), (0,)), ((), ())),
                        preferred_element_type=jnp.float32)
    o = o + b2_ref[...]
    n = jnp.sqrt(jnp.sum(o * o, axis=1, keepdims=True) + 1e-08)
    o_ref[...] = o / n


def _mlp(m, W1, b1, W2, b2):
    blk = 2048
    grid = (BATCH // blk,)
    return pl.pallas_call(
        _mlp_body,
        grid=grid,
        in_specs=[
            pl.BlockSpec((blk, EMBED_DIM), lambda i: (i, 0)),
            pl.BlockSpec((EMBED_DIM, HIDDEN_DIM), lambda i: (0, 0)),
            pl.BlockSpec((1, HIDDEN_DIM), lambda i: (0, 0)),
            pl.BlockSpec((HIDDEN_DIM, OUT_DIM), lambda i: (0, 0)),
            pl.BlockSpec((1, OUT_DIM), lambda i: (0, 0)),
        ],
        out_specs=pl.BlockSpec((blk, OUT_DIM), lambda i: (i, 0)),
        out_shape=jax.ShapeDtypeStruct((BATCH, OUT_DIM), jnp.float32),
    )(m, W1, b1.reshape(1, -1), W2, b2.reshape(1, -1))


def kernel(x, table, W1, b1, W2, b2):
    x_flat = x.reshape(-1).astype(jnp.int32)
    m = _pool(x_flat, table)
    return _mlp(m, W1, b1, W2, b2)
